# Initial kernel scaffold; baseline (speedup 1.0000x reference)
import jax
import jax.numpy as jnp
from jax.experimental import pallas as pl


def kernel(x, adj, Wl1, Wr1, att1, b1, Wl2, Wr2, att2, b2):
    raise NotImplementedError("write your pallas kernel here")


# speedup vs baseline: 1.0014x; 1.0014x over previous
import functools

import jax
import jax.numpy as jnp
from jax.experimental import pallas as pl

N = 10000
D = 128
H = 6
C = 128
DEG = 16
NEG_SLOPE = 0.2


def _mm_body(x_ref, w_ref, o_ref):
    o_ref[...] = jnp.dot(x_ref[...], w_ref[...],
                         preferred_element_type=jnp.float32)


@functools.partial(jax.jit, static_argnames=("block",))
def _matmul(x, w, block=512):
    n, k = x.shape
    m = w.shape[1]
    grid = (pl.cdiv(n, block),)
    return pl.pallas_call(
        _mm_body,
        grid=grid,
        in_specs=[
            pl.BlockSpec((block, k), lambda i: (i, 0)),
            pl.BlockSpec((k, m), lambda i: (0, 0)),
        ],
        out_specs=pl.BlockSpec((block, m), lambda i: (i, 0)),
        out_shape=jax.ShapeDtypeStruct((n, m), jnp.float32),
    )(x, w)


def _layer(x, src, dst, Wl, Wr, att, b):
    n = x.shape[0]
    xl = _matmul(x, Wl).reshape(n, H, C)
    xr = _matmul(x, Wr).reshape(n, H, C)
    e = xl[src] + xr[dst]
    e = jnp.where(e > 0, e, NEG_SLOPE * e)
    alpha = (e * att[None, :, :]).sum(-1)
    amax = jax.ops.segment_max(alpha, dst, num_segments=n + 1)
    amax = jnp.where(jnp.isfinite(amax), amax, 0.0)
    ex = jnp.exp(alpha - amax[dst])
    den = jax.ops.segment_sum(ex, dst, num_segments=n + 1)
    a = ex / (den[dst] + 1e-16)
    msg = xl[src] * a[:, :, None]
    out = jax.ops.segment_sum(msg, dst, num_segments=n + 1)
    return out[:n].mean(axis=1) + b


def kernel(x, adj, Wl1, Wr1, att1, b1, Wl2, Wr2, att2, b2):
    n = adj.shape[0]
    row, col = jnp.nonzero(adj, size=n * DEG, fill_value=(0, n))
    src, dst = row, col
    h = jax.nn.relu(_layer(x, src, dst, Wl1, Wr1, att1, b1))
    h = jax.nn.relu(_layer(h, src, dst, Wl2, Wr2, att2, b2))
    return h


# speedup vs baseline: 1.4842x; 1.4822x over previous
import functools

import jax
import jax.numpy as jnp
from jax import lax
from jax.experimental import pallas as pl
from jax.experimental.pallas import tpu as pltpu
from jax.experimental.pallas import tpu_sc as plsc

N = 10000
D = 128
H = 6
C = 128
DEG = 16
NEG_SLOPE = 0.2

NPAD = 10240
NW = 32
RPW = NPAD // NW
RCH = 8
NCHUNK = RPW // RCH
WORDS = 640
WREAL = 625


def _pack_body(a_ref, o_ref):
    j = pl.program_id(1)
    r2 = lax.broadcasted_iota(jnp.int32, (2048, 128), 0)
    c2 = lax.broadcasted_iota(jnp.int32, (2048, 128), 1)
    pow2 = lax.bitcast_convert_type(((r2 % 16) + 127) << 23, jnp.float32)
    p = jnp.where((r2 // 16) == c2, pow2, 0.0).astype(jnp.bfloat16)
    a = a_ref[...]
    col = j * 2048 + lax.broadcasted_iota(jnp.int32, a.shape, 1)
    a = jnp.where(col < N, a, 0.0).astype(jnp.bfloat16)
    acc = jnp.dot(a, p, preferred_element_type=jnp.float32)
    o_ref[...] = acc.astype(jnp.int32)


def _pack(adj):
    return pl.pallas_call(
        _pack_body,
        grid=(25, 5),
        in_specs=[pl.BlockSpec((400, 2048), lambda i, j: (i, j))],
        out_specs=pl.BlockSpec((400, 128), lambda i, j: (i, j)),
        out_shape=jax.ShapeDtypeStruct((N, WORDS), jnp.int32),
    )(adj)


def _expand_body(bits_hbm, edge_hbm, bits_v, ed_v, edpad, wbuf, wibuf, sem):
    cid = lax.axis_index("c")
    sid = lax.axis_index("s")
    wid = sid * 2 + cid
    row0 = wid * RPW
    iota = lax.iota(jnp.int32, 16)
    zeros16 = jnp.zeros((16,), jnp.int32)

    def chunk(g, _):
        base = row0 + g * RCH
        pltpu.async_copy(bits_hbm.at[pl.ds(base, RCH)], bits_v, sem).wait()

        def row(r, _):
            edpad[r, pl.ds(0, 16)] = zeros16
            edpad[r, pl.ds(16, 16)] = zeros16
            k = jnp.int32(0)
            for v in range(WORDS // 16):
                w = bits_v[r, pl.ds(v * 16, 16)]
                m = w != 0
                cs = plsc.cumsum(jnp.where(m, 1, 0))
                idx = jnp.where(m, k + cs - 1, 31)
                plsc.store_scatter(wbuf, [idx], w)
                plsc.store_scatter(wibuf, [idx], iota + v * 16)
                k = k + plsc.all_reduce_population_count(m)[0]

            def word(t, kk):
                wv = wbuf[pl.ds(t, 16)][0]
                wb = wibuf[pl.ds(t, 16)][0] * 16
                m2 = ((wv >> iota) & 1) != 0
                cs2 = plsc.cumsum(jnp.where(m2, 1, 0))
                idx2 = jnp.where(m2, kk + cs2 - 1, 31)
                plsc.store_scatter(edpad.at[r], [idx2], wb + iota + 1)
                return kk + plsc.all_reduce_population_count(m2)[0]

            lax.fori_loop(0, k, word, jnp.int32(0))
            ed_v[r, :] = edpad[r, pl.ds(0, 16)]
            return 0

        lax.fori_loop(0, RCH, row, 0)
        pltpu.async_copy(ed_v, edge_hbm.at[pl.ds(base, RCH)], sem).wait()
        return 0

    lax.fori_loop(0, NCHUNK, chunk, 0)


def _expand(bits):
    mesh = plsc.VectorSubcoreMesh(core_axis_name="c", subcore_axis_name="s",
                                  num_cores=2, num_subcores=16)
    f = pl.kernel(
        _expand_body,
        out_type=jax.ShapeDtypeStruct((NPAD, 16), jnp.int32),
        mesh=mesh,
        compiler_params=pltpu.CompilerParams(needs_layout_passes=False),
        scratch_types=[
            pltpu.VMEM((RCH, WORDS), jnp.int32),
            pltpu.VMEM((RCH, 16), jnp.int32),
            pltpu.VMEM((RCH, 32), jnp.int32),
            pltpu.VMEM((32,), jnp.int32),
            pltpu.VMEM((32,), jnp.int32),
            pltpu.SemaphoreType.DMA,
        ],
    )
    return f(bits)


def _mm_body(x_ref, w_ref, o_ref):
    o_ref[...] = jnp.dot(x_ref[...], w_ref[...],
                         preferred_element_type=jnp.float32)


def _matmul(x, w, block=512):
    n, k = x.shape
    m = w.shape[1]
    return pl.pallas_call(
        _mm_body,
        grid=(pl.cdiv(n, block),),
        in_specs=[
            pl.BlockSpec((block, k), lambda i: (i, 0)),
            pl.BlockSpec((k, m), lambda i: (0, 0)),
        ],
        out_specs=pl.BlockSpec((block, m), lambda i: (i, 0)),
        out_shape=jax.ShapeDtypeStruct((n, m), jnp.float32),
    )(x, w)


def _layer(x, src, dst, Wl, Wr, att, b):
    n = x.shape[0]
    xl = _matmul(x, Wl).reshape(n, H, C)
    xr = _matmul(x, Wr).reshape(n, H, C)
    e = xl[src] + xr[dst]
    e = jnp.where(e > 0, e, NEG_SLOPE * e)
    alpha = (e * att[None, :, :]).sum(-1)
    amax = jax.ops.segment_max(alpha, dst, num_segments=n + 1)
    amax = jnp.where(jnp.isfinite(amax), amax, 0.0)
    ex = jnp.exp(alpha - amax[dst])
    den = jax.ops.segment_sum(ex, dst, num_segments=n + 1)
    a = ex / (den[dst] + 1e-16)
    msg = xl[src] * a[:, :, None]
    out = jax.ops.segment_sum(msg, dst, num_segments=n + 1)
    return out[:n].mean(axis=1) + b


def kernel(x, adj, Wl1, Wr1, att1, b1, Wl2, Wr2, att2, b2):
    bits = _pack(adj)
    bits = jnp.pad(bits, ((0, NPAD - N), (0, 0)))
    ed = _expand(bits)
    edv = ed[:N].reshape(-1)
    src = jnp.repeat(jnp.arange(N, dtype=jnp.int32), DEG)
    dst = jnp.where(edv > 0, edv - 1, N).astype(jnp.int32)
    h = jax.nn.relu(_layer(x, src, dst, Wl1, Wr1, att1, b1))
    h = jax.nn.relu(_layer(h, src, dst, Wl2, Wr2, att2, b2))
    return h


# speedup vs baseline: 6.2933x; 4.2403x over previous
import functools

import jax
import jax.numpy as jnp
from jax import lax
from jax.experimental import pallas as pl
from jax.experimental.pallas import tpu as pltpu
from jax.experimental.pallas import tpu_sc as plsc

N = 10000
D = 128
H = 6
C = 128
DEG = 16
NEG_SLOPE = 0.2

NPAD = 10240
NW = 32
RPW = NPAD // NW
RCH = 8
NCHUNK = RPW // RCH
WORDS = 640
WREAL = 625


def _pack_body(a_ref, o_ref):
    j = pl.program_id(1)
    r2 = lax.broadcasted_iota(jnp.int32, (2048, 128), 0)
    c2 = lax.broadcasted_iota(jnp.int32, (2048, 128), 1)
    pow2 = lax.bitcast_convert_type(((r2 % 16) + 127) << 23, jnp.float32)
    p = jnp.where((r2 // 16) == c2, pow2, 0.0).astype(jnp.bfloat16)
    a = a_ref[...]
    col = j * 2048 + lax.broadcasted_iota(jnp.int32, a.shape, 1)
    a = jnp.where(col < N, a, 0.0).astype(jnp.bfloat16)
    acc = jnp.dot(a, p, preferred_element_type=jnp.float32)
    o_ref[...] = acc.astype(jnp.int32)


def _pack(adj):
    return pl.pallas_call(
        _pack_body,
        grid=(25, 5),
        in_specs=[pl.BlockSpec((400, 2048), lambda i, j: (i, j))],
        out_specs=pl.BlockSpec((400, 128), lambda i, j: (i, j)),
        out_shape=jax.ShapeDtypeStruct((N, WORDS), jnp.int32),
    )(adj)


def _expand_body(bits_hbm, edge_hbm, bits_v, ed_v, edpad, wbuf, wibuf, sem):
    cid = lax.axis_index("c")
    sid = lax.axis_index("s")
    wid = sid * 2 + cid
    row0 = wid * RPW
    iota = lax.iota(jnp.int32, 16)
    zeros16 = jnp.zeros((16,), jnp.int32)

    def chunk(g, _):
        base = row0 + g * RCH
        pltpu.async_copy(bits_hbm.at[pl.ds(base, RCH)], bits_v, sem).wait()

        def row(r, _):
            edpad[r, pl.ds(0, 16)] = zeros16
            edpad[r, pl.ds(16, 16)] = zeros16
            k = jnp.int32(0)
            for v in range(WORDS // 16):
                w = bits_v[r, pl.ds(v * 16, 16)]
                m = w != 0
                cs = plsc.cumsum(jnp.where(m, 1, 0))
                idx = jnp.where(m, k + cs - 1, 31)
                plsc.store_scatter(wbuf, [idx], w)
                plsc.store_scatter(wibuf, [idx], iota + v * 16)
                k = k + plsc.all_reduce_population_count(m)[0]

            def word(t, kk):
                wv = wbuf[pl.ds(t, 16)][0]
                wb = wibuf[pl.ds(t, 16)][0] * 16
                m2 = ((wv >> iota) & 1) != 0
                cs2 = plsc.cumsum(jnp.where(m2, 1, 0))
                idx2 = jnp.where(m2, kk + cs2 - 1, 31)
                plsc.store_scatter(edpad.at[r], [idx2], wb + iota + 1)
                return kk + plsc.all_reduce_population_count(m2)[0]

            lax.fori_loop(0, k, word, jnp.int32(0))
            ed_v[r, :] = edpad[r, pl.ds(0, 16)]
            return 0

        lax.fori_loop(0, RCH, row, 0)
        pltpu.async_copy(ed_v, edge_hbm.at[pl.ds(base, RCH)], sem).wait()
        return 0

    lax.fori_loop(0, NCHUNK, chunk, 0)


def _expand(bits):
    mesh = plsc.VectorSubcoreMesh(core_axis_name="c", subcore_axis_name="s",
                                  num_cores=2, num_subcores=16)
    f = pl.kernel(
        _expand_body,
        out_type=jax.ShapeDtypeStruct((NPAD, 16), jnp.int32),
        mesh=mesh,
        compiler_params=pltpu.CompilerParams(needs_layout_passes=False),
        scratch_types=[
            pltpu.VMEM((RCH, WORDS), jnp.int32),
            pltpu.VMEM((RCH, 16), jnp.int32),
            pltpu.VMEM((RCH, 32), jnp.int32),
            pltpu.VMEM((32,), jnp.int32),
            pltpu.VMEM((32,), jnp.int32),
            pltpu.SemaphoreType.DMA,
        ],
    )
    return f(bits)


HC = 128
HCP = 144
CH = 4
ECH = CH * 16
NCH2 = RPW // (2 * CH)


def _layer_body(xl_hbm, xls_hbm, xrs_hbm, ed_hbm, ss_hbm, num_hbm, den_hbm,
                ed0, ed1, idx0, idx1, sidx0, sidx1,
                xl0, xl1, xls0, xls1, xr0, xr1,
                ob, exb, ssv, den_v, num_sh, sem0, sem1):
    cid = lax.axis_index("c")
    sid = lax.axis_index("s")
    wid = sid * 2 + cid
    row0 = wid * RPW
    iota = lax.iota(jnp.int32, 16)
    zi = jnp.zeros((16,), jnp.int32)
    zf = jnp.zeros((16,), jnp.float32)
    ridx = [iota + r * 16 for r in range(CH)]

    def zrow(r, _):
        for cb in range(HC // 16):
            ob[r, pl.ds(cb * 16, 16)] = zf
        return 0

    def zden(i, _):
        den_v[pl.ds(i * 16, 16)] = zf
        return 0

    for h in range(H):
        pltpu.sync_copy(ss_hbm.at[h], ssv)
        lax.fori_loop(0, (NPAD + 16) // 16, zden, 0)
        lax.fori_loop(0, ECH, zrow, 0)
        plsc.subcore_barrier()
        for q in range(640 // ECH):
            pltpu.sync_copy(ob, num_sh.at[pl.ds(sid * 640 + q * ECH, ECH)])
        plsc.subcore_barrier()

        def stage(base, edb, idxb, sidxb, xlb, xlsb, xrb, sem):
            base = jnp.minimum(base, NPAD - CH)
            pltpu.sync_copy(ed_hbm.at[pl.ds(base, CH)], edb)
            for r in range(CH):
                edr = edb[r, :]
                im1 = jnp.maximum(edr - 1, zi)
                idxb[pl.ds(r * 16, 16)] = im1 + h * NPAD
                sidxb[pl.ds(r * 16, 16)] = im1
            gcp = pltpu.async_copy(xrs_hbm.at[idxb], xrb, sem)
            pltpu.sync_copy(xl_hbm.at[pl.ds(h * NPAD + base, CH)], xlb)
            pltpu.sync_copy(xls_hbm.at[pl.ds(h * NPAD + base, CH)], xlsb)
            return gcp

        def compute(base, edb, idxb, sidxb, xlb, xlsb, xrb, sem):
            pltpu.make_async_copy(xrs_hbm.at[idxb], xrb, sem).wait()

            def cbody(c, accs):
                cs = zf + ssv[pl.ds(c, 16)][0]
                colc = zi + c
                out = []
                for r in range(CH):
                    xv = plsc.load_gather(xrb, [ridx[r], colc])
                    v = xv + (zf + xlsb[r, pl.ds(c, 16)][0])
                    acc_l = accs[2 * r] + v
                    acc_s = accs[2 * r + 1] + cs * jnp.abs(v)
                    out.append(acc_l)
                    out.append(acc_s)
                return tuple(out)

            accs = lax.fori_loop(0, HC, cbody, tuple([zf] * (2 * CH)))
            for r in range(CH):
                alpha = accs[2 * r] * 0.6 + accs[2 * r + 1]
                edr = edb[r, :]
                ex = jnp.where(edr != 0, jnp.exp(alpha), zf)
                exb[r, :] = ex
                dix = jnp.where(edr != 0, edr - 1, zi + NPAD)
                plsc.addupdate_scatter(den_v, [dix], ex)

            def mrow(r, _):
                exr = exb[r, :]
                for jj in range(16):
                    s = zf + exr[jj]
                    e = r * 16 + jj
                    for cb in range(8):
                        ob[e, pl.ds(cb * 16, 16)] = s * xlb[r, pl.ds(cb * 16, 16)]
                return 0

            lax.fori_loop(0, CH, mrow, 0)
            pltpu.sync_copy(ob, num_sh.at[sidxb], add=True)

        b0 = (ed0, idx0, sidx0, xl0, xls0, xr0, sem0)
        b1 = (ed1, idx1, sidx1, xl1, xls1, xr1, sem1)
        stage(row0, *b0)

        def pair(t, _):
            base = row0 + 2 * t * CH
            stage(base + CH, *b1)
            compute(base, *b0)
            stage(base + 2 * CH, *b0)
            compute(base + CH, *b1)
            return 0

        lax.fori_loop(0, NCH2, pair, 0)
        pltpu.make_async_copy(xrs_hbm.at[idx0], xr0, sem0).wait()
        plsc.subcore_barrier()
        pltpu.sync_copy(num_sh.at[pl.ds(sid * 640, 640)],
                        num_hbm.at[cid * H + h, pl.ds(sid * 640, 640)])
        pltpu.sync_copy(den_v.at[pl.ds(0, NPAD)], den_hbm.at[h, wid])
    plsc.subcore_barrier()


def _sc_layer(xl, xls, xrs, ed, ss):
    mesh = plsc.VectorSubcoreMesh(core_axis_name="c", subcore_axis_name="s",
                                  num_cores=2, num_subcores=16)
    f = pl.kernel(
        _layer_body,
        out_type=[jax.ShapeDtypeStruct((2 * H, NPAD, HC), jnp.float32),
                  jax.ShapeDtypeStruct((H, NW, NPAD), jnp.float32)],
        mesh=mesh,
        compiler_params=pltpu.CompilerParams(needs_layout_passes=False),
        scratch_types=[
            pltpu.VMEM((CH, 16), jnp.int32),
            pltpu.VMEM((CH, 16), jnp.int32),
            pltpu.VMEM((ECH,), jnp.int32),
            pltpu.VMEM((ECH,), jnp.int32),
            pltpu.VMEM((ECH,), jnp.int32),
            pltpu.VMEM((ECH,), jnp.int32),
            pltpu.VMEM((CH, HC), jnp.float32),
            pltpu.VMEM((CH, HC), jnp.float32),
            pltpu.VMEM((CH, HCP), jnp.float32),
            pltpu.VMEM((CH, HCP), jnp.float32),
            pltpu.VMEM((ECH, HC), jnp.float32),
            pltpu.VMEM((ECH, HC), jnp.float32),
            pltpu.VMEM((ECH, HC), jnp.float32),
            pltpu.VMEM((CH, 16), jnp.float32),
            pltpu.VMEM((HCP,), jnp.float32),
            pltpu.VMEM((NPAD + 16,), jnp.float32),
            pltpu.VMEM_SHARED((NPAD, HC), jnp.float32),
            pltpu.SemaphoreType.DMA,
            pltpu.SemaphoreType.DMA,
        ],
    )
    return f(xl, xls, xrs, ed, ss)


def _mm3_body(x_ref, w_ref, xl_ref, xls_ref, xrs_ref):
    y = jnp.dot(x_ref[...], w_ref[...].reshape(D, 3 * HC),
                preferred_element_type=jnp.float32)
    xl_ref[...] = y[:, :HC]
    xls_ref[...] = jnp.concatenate(
        [y[:, HC:2 * HC], jnp.zeros((y.shape[0], HCP - HC), jnp.float32)], axis=1)
    xrs_ref[...] = y[:, 2 * HC:]


def _mm3(x, w3):
    nb = NPAD // 512
    return pl.pallas_call(
        _mm3_body,
        grid=(nb, H),
        in_specs=[
            pl.BlockSpec((512, D), lambda i, h: (i, 0)),
            pl.BlockSpec((1, D, 3 * HC), lambda i, h: (h, 0, 0)),
        ],
        out_specs=[
            pl.BlockSpec((512, HC), lambda i, h: (h * (NPAD // 512) + i, 0)),
            pl.BlockSpec((512, HCP), lambda i, h: (h * (NPAD // 512) + i, 0)),
            pl.BlockSpec((512, HC), lambda i, h: (h * (NPAD // 512) + i, 0)),
        ],
        out_shape=[
            jax.ShapeDtypeStruct((H * NPAD, HC), jnp.float32),
            jax.ShapeDtypeStruct((H * NPAD, HCP), jnp.float32),
            jax.ShapeDtypeStruct((H * NPAD, HC), jnp.float32),
        ],
    )(x, w3)


def _fin_body(num_ref, den_ref, b_ref, o_ref):
    v = num_ref[...]
    msg = v[:H] + v[H:]
    den = jnp.sum(den_ref[...], axis=1)
    a = msg / (den[:, :, None] + 1e-16)
    o_ref[...] = jax.nn.relu(jnp.mean(a, axis=0) + b_ref[...])


def _finish(num, den, b):
    nb = NPAD // 512
    return pl.pallas_call(
        _fin_body,
        grid=(nb,),
        in_specs=[
            pl.BlockSpec((2 * H, 512, HC), lambda i: (0, i, 0)),
            pl.BlockSpec((H, NW, 512), lambda i: (0, 0, i)),
            pl.BlockSpec((1, HC), lambda i: (0, 0)),
        ],
        out_specs=pl.BlockSpec((512, HC), lambda i: (i, 0)),
        out_shape=jax.ShapeDtypeStruct((NPAD, HC), jnp.float32),
    )(num, den, b.reshape(1, HC))


def _mm_body(x_ref, w_ref, o_ref):
    o_ref[...] = jnp.dot(x_ref[...], w_ref[...],
                         preferred_element_type=jnp.float32)


def _matmul(x, w, block=512):
    n, k = x.shape
    m = w.shape[1]
    return pl.pallas_call(
        _mm_body,
        grid=(pl.cdiv(n, block),),
        in_specs=[
            pl.BlockSpec((block, k), lambda i: (i, 0)),
            pl.BlockSpec((k, m), lambda i: (0, 0)),
        ],
        out_specs=pl.BlockSpec((block, m), lambda i: (i, 0)),
        out_shape=jax.ShapeDtypeStruct((n, m), jnp.float32),
    )(x, w)


def _layer(x, src, dst, Wl, Wr, att, b):
    n = x.shape[0]
    xl = _matmul(x, Wl).reshape(n, H, C)
    xr = _matmul(x, Wr).reshape(n, H, C)
    e = xl[src] + xr[dst]
    e = jnp.where(e > 0, e, NEG_SLOPE * e)
    alpha = (e * att[None, :, :]).sum(-1)
    amax = jax.ops.segment_max(alpha, dst, num_segments=n + 1)
    amax = jnp.where(jnp.isfinite(amax), amax, 0.0)
    ex = jnp.exp(alpha - amax[dst])
    den = jax.ops.segment_sum(ex, dst, num_segments=n + 1)
    a = ex / (den[dst] + 1e-16)
    msg = xl[src] * a[:, :, None]
    out = jax.ops.segment_sum(msg, dst, num_segments=n + 1)
    return out[:n].mean(axis=1) + b


def _prep_w(Wl, Wr, att):
    attf = att.reshape(H * C)
    w3 = jnp.stack([Wl.reshape(D, H, C),
                    (Wl * attf).reshape(D, H, C),
                    (Wr * attf).reshape(D, H, C)], axis=2)
    w3 = w3.reshape(D, H, 3 * C).swapaxes(0, 1)
    ss = jnp.pad(0.4 * jnp.sign(att), ((0, 0), (0, HCP - HC)))
    return w3, ss


def kernel(x, adj, Wl1, Wr1, att1, b1, Wl2, Wr2, att2, b2):
    bits = _pack(adj)
    bits = jnp.pad(bits, ((0, NPAD - N), (0, 0)))
    ed = _expand(bits)
    xp = jnp.pad(x, ((0, NPAD - N), (0, 0)))
    w31, ss1 = _prep_w(Wl1, Wr1, att1)
    xl, xls, xrs = _mm3(xp, w31)
    num, den = _sc_layer(xl, xls, xrs, ed, ss1)
    h1 = _finish(num, den, b1)
    w32, ss2 = _prep_w(Wl2, Wr2, att2)
    xl2, xls2, xrs2 = _mm3(h1, w32)
    num2, den2 = _sc_layer(xl2, xls2, xrs2, ed, ss2)
    h2 = _finish(num2, den2, b2)
    return h2[:N]


# speedup vs baseline: 7.6410x; 1.2141x over previous
import functools

import jax
import jax.numpy as jnp
from jax import lax
from jax.experimental import pallas as pl
from jax.experimental.pallas import tpu as pltpu
from jax.experimental.pallas import tpu_sc as plsc

N = 10000
D = 128
H = 6
C = 128
DEG = 16
NEG_SLOPE = 0.2

NPAD = 10240
NW = 32
RPW = NPAD // NW
RCH = 8
NCHUNK = RPW // RCH
WORDS = 640
WREAL = 625


def _pack_body(a_ref, o_ref):
    j = pl.program_id(1)
    r2 = lax.broadcasted_iota(jnp.int32, (2048, 128), 0)
    c2 = lax.broadcasted_iota(jnp.int32, (2048, 128), 1)
    pow2 = lax.bitcast_convert_type(((r2 % 16) + 127) << 23, jnp.float32)
    p = jnp.where((r2 // 16) == c2, pow2, 0.0).astype(jnp.bfloat16)
    a = a_ref[...]
    col = j * 2048 + lax.broadcasted_iota(jnp.int32, a.shape, 1)
    a = jnp.where(col < N, a, 0.0).astype(jnp.bfloat16)
    acc = jnp.dot(a, p, preferred_element_type=jnp.float32)
    o_ref[...] = acc.astype(jnp.int32)


def _pack(adj):
    return pl.pallas_call(
        _pack_body,
        grid=(25, 5),
        in_specs=[pl.BlockSpec((400, 2048), lambda i, j: (i, j))],
        out_specs=pl.BlockSpec((400, 128), lambda i, j: (i, j)),
        out_shape=jax.ShapeDtypeStruct((N, WORDS), jnp.int32),
    )(adj)


def _expand_body(bits_hbm, edge_hbm, bits_v, ed_v, edpad, wbuf, wibuf, sem):
    cid = lax.axis_index("c")
    sid = lax.axis_index("s")
    wid = sid * 2 + cid
    row0 = wid * RPW
    iota = lax.iota(jnp.int32, 16)
    zeros16 = jnp.zeros((16,), jnp.int32)

    def chunk(g, _):
        base = row0 + g * RCH
        pltpu.async_copy(bits_hbm.at[pl.ds(base, RCH)], bits_v, sem).wait()

        def row(r, _):
            edpad[r, pl.ds(0, 16)] = zeros16
            edpad[r, pl.ds(16, 16)] = zeros16
            k = jnp.int32(0)
            for v in range(WORDS // 16):
                w = bits_v[r, pl.ds(v * 16, 16)]
                m = w != 0
                cs = plsc.cumsum(jnp.where(m, 1, 0))
                idx = jnp.where(m, k + cs - 1, 31)
                plsc.store_scatter(wbuf, [idx], w)
                plsc.store_scatter(wibuf, [idx], iota + v * 16)
                k = k + plsc.all_reduce_population_count(m)[0]

            def word(t, kk):
                wv = wbuf[pl.ds(t, 16)][0]
                wb = wibuf[pl.ds(t, 16)][0] * 16
                m2 = ((wv >> iota) & 1) != 0
                cs2 = plsc.cumsum(jnp.where(m2, 1, 0))
                idx2 = jnp.where(m2, kk + cs2 - 1, 31)
                plsc.store_scatter(edpad.at[r], [idx2], wb + iota + 1)
                return kk + plsc.all_reduce_population_count(m2)[0]

            lax.fori_loop(0, k, word, jnp.int32(0))
            ed_v[r, :] = edpad[r, pl.ds(0, 16)]
            return 0

        lax.fori_loop(0, RCH, row, 0)
        pltpu.async_copy(ed_v, edge_hbm.at[pl.ds(base, RCH)], sem).wait()
        return 0

    lax.fori_loop(0, NCHUNK, chunk, 0)


def _expand(bits):
    mesh = plsc.VectorSubcoreMesh(core_axis_name="c", subcore_axis_name="s",
                                  num_cores=2, num_subcores=16)
    f = pl.kernel(
        _expand_body,
        out_type=jax.ShapeDtypeStruct((NPAD, 16), jnp.int32),
        mesh=mesh,
        compiler_params=pltpu.CompilerParams(needs_layout_passes=False),
        scratch_types=[
            pltpu.VMEM((RCH, WORDS), jnp.int32),
            pltpu.VMEM((RCH, 16), jnp.int32),
            pltpu.VMEM((RCH, 32), jnp.int32),
            pltpu.VMEM((32,), jnp.int32),
            pltpu.VMEM((32,), jnp.int32),
            pltpu.SemaphoreType.DMA,
        ],
    )
    return f(bits)


HC = 128
HCP = 144
CH = 4
ECH = CH * 16
NCH2 = RPW // (2 * CH)


def _layer_body(xl_hbm, xls_hbm, xrs_hbm, ed_hbm, ss_hbm, num_hbm, den_hbm,
                ed_all, idx0, idx1, sidx0, sidx1,
                xl0, xl1, xls0, xls1, xr0, xr1,
                ob, exb, ssv, den_v, num_sh, sem0, sem1, xsem0, xsem1):
    cid = lax.axis_index("c")
    sid = lax.axis_index("s")
    wid = sid * 2 + cid
    row0 = wid * RPW
    iota = lax.iota(jnp.int32, 16)
    zi = jnp.zeros((16,), jnp.int32)
    zf = jnp.zeros((16,), jnp.float32)
    ridx = [iota + r * 16 for r in range(CH)]

    def zrow(r, _):
        for cb in range(HC // 16):
            ob[r, pl.ds(cb * 16, 16)] = zf
        return 0

    def zden(i, _):
        den_v[pl.ds(i * 16, 16)] = zf
        return 0

    pltpu.sync_copy(ed_hbm.at[pl.ds(wid * (RPW // 8), RPW // 8)], ed_all)

    def edrow(v):
        return ed_all[v >> 3, pl.ds((v & 7) * 16, 16)]

    for h in range(H):
        pltpu.sync_copy(ss_hbm.at[h], ssv)
        lax.fori_loop(0, (NPAD + 16) // 16, zden, 0)
        lax.fori_loop(0, ECH, zrow, 0)
        plsc.subcore_barrier()
        for q in range(640 // ECH):
            pltpu.sync_copy(ob, num_sh.at[pl.ds(sid * 640 + q * ECH, ECH)])
        plsc.subcore_barrier()

        def stage(base, idxb, sidxb, xlb, xlsb, xrb, sem, xsem):
            base = jnp.minimum(base, row0 + RPW - CH)
            co = base - row0
            for r in range(CH):
                edr = edrow(co + r)
                im1 = jnp.maximum(edr - 1, zi)
                idxb[pl.ds(r * 16, 16)] = im1 + h * NPAD
                sidxb[pl.ds(r * 16, 16)] = im1
            pltpu.async_copy(xrs_hbm.at[idxb], xrb, sem)
            pltpu.async_copy(xl_hbm.at[pl.ds(h * NPAD + base, CH)], xlb, xsem)
            pltpu.async_copy(xls_hbm.at[pl.ds(h * NPAD + base, CH)], xlsb, xsem)

        def compute(base, idxb, sidxb, xlb, xlsb, xrb, sem, xsem):
            base = jnp.minimum(base, row0 + RPW - CH)
            co = base - row0
            pltpu.make_async_copy(xrs_hbm.at[idxb], xrb, sem).wait()
            pltpu.make_async_copy(
                xl_hbm.at[pl.ds(h * NPAD + base, CH)], xlb, xsem).wait()
            pltpu.make_async_copy(
                xls_hbm.at[pl.ds(h * NPAD + base, CH)], xlsb, xsem).wait()

            def cbody(c, accs):
                cs = zf + ssv[pl.ds(c, 16)][0]
                colc = zi + c
                out = []
                for r in range(CH):
                    xv = plsc.load_gather(xrb, [ridx[r], colc])
                    v = xv + (zf + xlsb[r, pl.ds(c, 16)][0])
                    acc_l = accs[2 * r] + v
                    acc_s = accs[2 * r + 1] + cs * jnp.abs(v)
                    out.append(acc_l)
                    out.append(acc_s)
                return tuple(out)

            accs = lax.fori_loop(0, HC, cbody, tuple([zf] * (2 * CH)))
            for r in range(CH):
                alpha = accs[2 * r] * 0.6 + accs[2 * r + 1]
                edr = edrow(co + r)
                ex = jnp.where(edr != 0, jnp.exp(alpha), zf)
                exb[r, :] = ex
                dix = jnp.where(edr != 0, edr - 1, zi + NPAD)
                plsc.addupdate_scatter(den_v, [dix], ex)

            def mrow(r, _):
                exr = exb[r, :]
                for jj in range(16):
                    s = zf + exr[jj]
                    e = r * 16 + jj
                    for cb in range(8):
                        ob[e, pl.ds(cb * 16, 16)] = s * xlb[r, pl.ds(cb * 16, 16)]
                return 0

            lax.fori_loop(0, CH, mrow, 0)
            pltpu.sync_copy(ob, num_sh.at[sidxb], add=True)

        b0 = (idx0, sidx0, xl0, xls0, xr0, sem0, xsem0)
        b1 = (idx1, sidx1, xl1, xls1, xr1, sem1, xsem1)
        stage(row0, *b0)

        def pair(t, _):
            base = row0 + 2 * t * CH
            stage(base + CH, *b1)
            compute(base, *b0)
            stage(base + 2 * CH, *b0)
            compute(base + CH, *b1)
            return 0

        lax.fori_loop(0, NCH2, pair, 0)
        pltpu.make_async_copy(xrs_hbm.at[idx0], xr0, sem0).wait()
        pltpu.make_async_copy(
            xl_hbm.at[pl.ds(h * NPAD + row0, CH)], xl0, xsem0).wait()
        pltpu.make_async_copy(
            xls_hbm.at[pl.ds(h * NPAD + row0, CH)], xls0, xsem0).wait()
        plsc.subcore_barrier()
        pltpu.sync_copy(num_sh.at[pl.ds(sid * 640, 640)],
                        num_hbm.at[cid * H + h, pl.ds(sid * 640, 640)])
        pltpu.sync_copy(den_v.at[pl.ds(0, NPAD)], den_hbm.at[h, wid])
    plsc.subcore_barrier()


def _sc_layer(xl, xls, xrs, ed, ss):
    mesh = plsc.VectorSubcoreMesh(core_axis_name="c", subcore_axis_name="s",
                                  num_cores=2, num_subcores=16)
    f = pl.kernel(
        _layer_body,
        out_type=[jax.ShapeDtypeStruct((2 * H, NPAD, HC), jnp.float32),
                  jax.ShapeDtypeStruct((H, NW, NPAD), jnp.float32)],
        mesh=mesh,
        compiler_params=pltpu.CompilerParams(needs_layout_passes=False),
        scratch_types=[
            pltpu.VMEM((RPW // 8, 128), jnp.int32),
            pltpu.VMEM((ECH,), jnp.int32),
            pltpu.VMEM((ECH,), jnp.int32),
            pltpu.VMEM((ECH,), jnp.int32),
            pltpu.VMEM((ECH,), jnp.int32),
            pltpu.VMEM((CH, HC), jnp.float32),
            pltpu.VMEM((CH, HC), jnp.float32),
            pltpu.VMEM((CH, HCP), jnp.float32),
            pltpu.VMEM((CH, HCP), jnp.float32),
            pltpu.VMEM((ECH, HC), jnp.float32),
            pltpu.VMEM((ECH, HC), jnp.float32),
            pltpu.VMEM((ECH, HC), jnp.float32),
            pltpu.VMEM((CH, 16), jnp.float32),
            pltpu.VMEM((HCP,), jnp.float32),
            pltpu.VMEM((NPAD + 16,), jnp.float32),
            pltpu.VMEM_SHARED((NPAD, HC), jnp.float32),
            pltpu.SemaphoreType.DMA,
            pltpu.SemaphoreType.DMA,
            pltpu.SemaphoreType.DMA,
            pltpu.SemaphoreType.DMA,
        ],
    )
    return f(xl, xls, xrs, ed.reshape(NPAD // 8, 8 * 16), ss)


def _mm3_body(x_ref, w_ref, xl_ref, xls_ref, xrs_ref):
    y = jnp.dot(x_ref[...], w_ref[...].reshape(D, 3 * HC),
                preferred_element_type=jnp.float32)
    xl_ref[...] = y[:, :HC]
    xls_ref[...] = jnp.concatenate(
        [y[:, HC:2 * HC], jnp.zeros((y.shape[0], HCP - HC), jnp.float32)], axis=1)
    xrs_ref[...] = y[:, 2 * HC:]


def _mm3(x, w3):
    nb = NPAD // 512
    return pl.pallas_call(
        _mm3_body,
        grid=(nb, H),
        in_specs=[
            pl.BlockSpec((512, D), lambda i, h: (i, 0)),
            pl.BlockSpec((1, D, 3 * HC), lambda i, h: (h, 0, 0)),
        ],
        out_specs=[
            pl.BlockSpec((512, HC), lambda i, h: (h * (NPAD // 512) + i, 0)),
            pl.BlockSpec((512, HCP), lambda i, h: (h * (NPAD // 512) + i, 0)),
            pl.BlockSpec((512, HC), lambda i, h: (h * (NPAD // 512) + i, 0)),
        ],
        out_shape=[
            jax.ShapeDtypeStruct((H * NPAD, HC), jnp.float32),
            jax.ShapeDtypeStruct((H * NPAD, HCP), jnp.float32),
            jax.ShapeDtypeStruct((H * NPAD, HC), jnp.float32),
        ],
    )(x, w3)


def _fin_body(num_ref, den_ref, b_ref, o_ref):
    v = num_ref[...]
    msg = v[:H] + v[H:]
    den = jnp.sum(den_ref[...], axis=1)
    a = msg / (den[:, :, None] + 1e-16)
    o_ref[...] = jax.nn.relu(jnp.mean(a, axis=0) + b_ref[...])


def _finish(num, den, b):
    nb = NPAD // 512
    return pl.pallas_call(
        _fin_body,
        grid=(nb,),
        in_specs=[
            pl.BlockSpec((2 * H, 512, HC), lambda i: (0, i, 0)),
            pl.BlockSpec((H, NW, 512), lambda i: (0, 0, i)),
            pl.BlockSpec((1, HC), lambda i: (0, 0)),
        ],
        out_specs=pl.BlockSpec((512, HC), lambda i: (i, 0)),
        out_shape=jax.ShapeDtypeStruct((NPAD, HC), jnp.float32),
    )(num, den, b.reshape(1, HC))


def _mm_body(x_ref, w_ref, o_ref):
    o_ref[...] = jnp.dot(x_ref[...], w_ref[...],
                         preferred_element_type=jnp.float32)


def _matmul(x, w, block=512):
    n, k = x.shape
    m = w.shape[1]
    return pl.pallas_call(
        _mm_body,
        grid=(pl.cdiv(n, block),),
        in_specs=[
            pl.BlockSpec((block, k), lambda i: (i, 0)),
            pl.BlockSpec((k, m), lambda i: (0, 0)),
        ],
        out_specs=pl.BlockSpec((block, m), lambda i: (i, 0)),
        out_shape=jax.ShapeDtypeStruct((n, m), jnp.float32),
    )(x, w)


def _layer(x, src, dst, Wl, Wr, att, b):
    n = x.shape[0]
    xl = _matmul(x, Wl).reshape(n, H, C)
    xr = _matmul(x, Wr).reshape(n, H, C)
    e = xl[src] + xr[dst]
    e = jnp.where(e > 0, e, NEG_SLOPE * e)
    alpha = (e * att[None, :, :]).sum(-1)
    amax = jax.ops.segment_max(alpha, dst, num_segments=n + 1)
    amax = jnp.where(jnp.isfinite(amax), amax, 0.0)
    ex = jnp.exp(alpha - amax[dst])
    den = jax.ops.segment_sum(ex, dst, num_segments=n + 1)
    a = ex / (den[dst] + 1e-16)
    msg = xl[src] * a[:, :, None]
    out = jax.ops.segment_sum(msg, dst, num_segments=n + 1)
    return out[:n].mean(axis=1) + b


def _prep_w(Wl, Wr, att):
    attf = att.reshape(H * C)
    w3 = jnp.stack([Wl.reshape(D, H, C),
                    (Wl * attf).reshape(D, H, C),
                    (Wr * attf).reshape(D, H, C)], axis=2)
    w3 = w3.reshape(D, H, 3 * C).swapaxes(0, 1)
    ss = jnp.pad(0.4 * jnp.sign(att), ((0, 0), (0, HCP - HC)))
    return w3, ss


def kernel(x, adj, Wl1, Wr1, att1, b1, Wl2, Wr2, att2, b2):
    bits = _pack(adj)
    bits = jnp.pad(bits, ((0, NPAD - N), (0, 0)))
    ed = _expand(bits)
    xp = jnp.pad(x, ((0, NPAD - N), (0, 0)))
    w31, ss1 = _prep_w(Wl1, Wr1, att1)
    xl, xls, xrs = _mm3(xp, w31)
    num, den = _sc_layer(xl, xls, xrs, ed, ss1)
    h1 = _finish(num, den, b1)
    w32, ss2 = _prep_w(Wl2, Wr2, att2)
    xl2, xls2, xrs2 = _mm3(h1, w32)
    num2, den2 = _sc_layer(xl2, xls2, xrs2, ed, ss2)
    h2 = _finish(num2, den2, b2)
    return h2[:N]
